# Initial kernel scaffold; baseline (speedup 1.0000x reference)
import jax
import jax.numpy as jnp
from jax.experimental import pallas as pl


def kernel(x, edge_index, label, Wl, bl, Wr, br, weight):
    raise NotImplementedError("write your pallas kernel here")


# speedup vs baseline: 16.7818x; 16.7818x over previous
import math

import jax
import jax.numpy as jnp
from jax import lax
from jax.experimental import pallas as pl
from jax.experimental.pallas import tpu as pltpu
from jax.experimental.pallas import tpu_sc as plsc

_N = 10000
_E = 320000
_D_IN = 128
_D_OUT = 16
_N_LABELS = 32
_S = 64.0
_M = 0.1
_COS_M = math.cos(_M)
_SIN_M = math.sin(_M)
_TH = math.cos(math.pi - _M)
_MM = math.sin(math.pi - _M) * _M

_NC = 2
_NS = 16
_NW = _NC * _NS
_EPW = _E // _NW
_C = 80
_K = _EPW // _C
_NBUF = 5
_G = _K // _NBUF
_RPS = _N // _NS

_ROWS_TC = 2000


def _project(x, wlT, wrT, bl2, br2):
    grid = (_N // _ROWS_TC,)

    def body(x_ref, wl_ref, wr_ref, bl_ref, br_ref, y_ref, z_ref):
        xb = x_ref[...]
        y_ref[...] = jnp.dot(xb, wl_ref[...], preferred_element_type=jnp.float32)
        z_ref[...] = (
            jnp.dot(xb, wr_ref[...], preferred_element_type=jnp.float32)
            + bl_ref[...] + br_ref[...]
        )

    return pl.pallas_call(
        body,
        grid=grid,
        in_specs=[
            pl.BlockSpec((_ROWS_TC, _D_IN), lambda i: (i, 0)),
            pl.BlockSpec((_D_IN, _D_OUT), lambda i: (0, 0)),
            pl.BlockSpec((_D_IN, _D_OUT), lambda i: (0, 0)),
            pl.BlockSpec((1, _D_OUT), lambda i: (0, 0)),
            pl.BlockSpec((1, _D_OUT), lambda i: (0, 0)),
        ],
        out_specs=[
            pl.BlockSpec((_ROWS_TC, _D_OUT), lambda i: (i, 0)),
            pl.BlockSpec((_ROWS_TC, _D_OUT), lambda i: (i, 0)),
        ],
        out_shape=[
            jax.ShapeDtypeStruct((_N, _D_OUT), jnp.float32),
            jax.ShapeDtypeStruct((_N, _D_OUT), jnp.float32),
        ],
    )(x, wlT, wrT, bl2, br2)


def _segment_sum_sc(y, src, dst):
    mesh = plsc.VectorSubcoreMesh(
        core_axis_name="c", subcore_axis_name="s",
        num_cores=_NC, num_subcores=_NS,
    )

    @pl.kernel(
        out_type=[
            jax.ShapeDtypeStruct((_NC, _N, _D_OUT), jnp.float32),
            jax.ShapeDtypeStruct((_NC, _N, _D_OUT), jnp.float32),
        ],
        mesh=mesh,
        scratch_types=[
            pltpu.VMEM((_K, _C), jnp.int32),
            pltpu.VMEM((_K, _C), jnp.int32),
            pltpu.VMEM((_NBUF, _C, _D_OUT), jnp.float32),
            pltpu.VMEM((_C, _D_OUT), jnp.float32),
            pltpu.VMEM((_RPS, _D_OUT), jnp.float32),
            pltpu.VMEM_SHARED((_N, _D_OUT), jnp.float32),
            pltpu.VMEM_SHARED((_N, _D_OUT), jnp.float32),
            pltpu.SemaphoreType.DMA,
            pltpu.SemaphoreType.DMA,
        ],
        compiler_params=pltpu.CompilerParams(use_tc_tiling_on_sc=False),
    )
    def seg(y_hbm, src_hbm, dst_hbm, agg_out, cnt_out,
            src_v, dst_v, rows_v, ones_v, tmp_v, agg_s, cnt_s, gsem, ssem):
        cid = lax.axis_index("c")
        sid = lax.axis_index("s")
        w = cid * _NS + sid

        ones16 = jnp.ones((_D_OUT,), jnp.float32)
        zeros16 = jnp.zeros((_D_OUT,), jnp.float32)

        def fill_ones(i, carry):
            ones_v[i, :] = ones16
            return carry
        lax.fori_loop(0, _C, fill_ones, 0)

        def fill_zero(i, carry):
            tmp_v[i, :] = zeros16
            return carry
        lax.fori_loop(0, _RPS, fill_zero, 0)

        row0 = sid * _RPS
        pltpu.sync_copy(tmp_v, agg_s.at[pl.ds(row0, _RPS)])
        pltpu.sync_copy(tmp_v, cnt_s.at[pl.ds(row0, _RPS)])

        pltpu.sync_copy(src_hbm.at[w], src_v)
        pltpu.sync_copy(dst_hbm.at[w], dst_v)

        plsc.subcore_barrier()

        def group(g, carry):
            base = g * _NBUF
            descs = []
            for b in range(_NBUF):
                descs.append(
                    pltpu.async_copy(y_hbm.at[src_v.at[base + b]],
                                     rows_v.at[b], gsem))
            for d in descs:
                d.wait()
            sdescs = []
            for b in range(_NBUF):
                sdescs.append(
                    pltpu.async_copy(rows_v.at[b],
                                     agg_s.at[dst_v.at[base + b]],
                                     ssem, add=True))
                sdescs.append(
                    pltpu.async_copy(ones_v,
                                     cnt_s.at[dst_v.at[base + b]],
                                     ssem, add=True))
            for d in sdescs:
                d.wait()
            return carry
        lax.fori_loop(0, _G, group, 0)

        plsc.subcore_barrier()

        pltpu.sync_copy(agg_s.at[pl.ds(row0, _RPS)], tmp_v)
        pltpu.sync_copy(tmp_v, agg_out.at[cid].at[pl.ds(row0, _RPS)])
        pltpu.sync_copy(cnt_s.at[pl.ds(row0, _RPS)], tmp_v)
        pltpu.sync_copy(tmp_v, cnt_out.at[cid].at[pl.ds(row0, _RPS)])

    return seg(y, src, dst)


def _head(agg, cnt, z, label2, weight):
    grid = (_N // _ROWS_TC,)

    def body(a_ref, c_ref, z_ref, l_ref, w_ref, feat_ref, out_ref):
        aggs = a_ref[0] + a_ref[1]
        cnts = c_ref[0, :, 0:1] + c_ref[1, :, 0:1]
        mean = aggs / jnp.maximum(cnts, 1.0)
        h = mean + z_ref[...]
        feat_ref[...] = h
        hr = jnp.maximum(h, 0.0)
        nrm = jnp.sqrt(jnp.sum(hr * hr, axis=1, keepdims=True))
        xn = hr / jnp.maximum(nrm, 1e-12)
        wv = w_ref[...]
        wn = wv / jnp.maximum(
            jnp.sqrt(jnp.sum(wv * wv, axis=1, keepdims=True)), 1e-12)
        cos = lax.dot_general(xn, wn, (((1,), (1,)), ((), ())),
                              preferred_element_type=jnp.float32)
        sine = jnp.sqrt(jnp.clip(1.0 - cos * cos, 0.0, 1.0))
        phi = cos * _COS_M - sine * _SIN_M
        phi = jnp.where(cos > _TH, phi, cos - _MM)
        onehot = lax.broadcasted_iota(jnp.int32, cos.shape, 1) == l_ref[...]
        out_ref[...] = jnp.where(onehot, phi, cos) * _S

    return pl.pallas_call(
        body,
        grid=grid,
        in_specs=[
            pl.BlockSpec((_NC, _ROWS_TC, _D_OUT), lambda i: (0, i, 0)),
            pl.BlockSpec((_NC, _ROWS_TC, _D_OUT), lambda i: (0, i, 0)),
            pl.BlockSpec((_ROWS_TC, _D_OUT), lambda i: (i, 0)),
            pl.BlockSpec((_ROWS_TC, 1), lambda i: (i, 0)),
            pl.BlockSpec((_N_LABELS, _D_OUT), lambda i: (0, 0)),
        ],
        out_specs=[
            pl.BlockSpec((_ROWS_TC, _D_OUT), lambda i: (i, 0)),
            pl.BlockSpec((_ROWS_TC, _N_LABELS), lambda i: (i, 0)),
        ],
        out_shape=[
            jax.ShapeDtypeStruct((_N, _D_OUT), jnp.float32),
            jax.ShapeDtypeStruct((_N, _N_LABELS), jnp.float32),
        ],
    )(agg, cnt, z, label2, weight)


def kernel(x, edge_index, label, Wl, bl, Wr, br, weight):
    y, z = _project(x, Wl.T, Wr.T, bl.reshape(1, _D_OUT), br.reshape(1, _D_OUT))
    src = edge_index[0].reshape(_NW, _K, _C)
    dst = edge_index[1].reshape(_NW, _K, _C)
    agg, cnt = _segment_sum_sc(y, src, dst)
    feat, out = _head(agg, cnt, z, label.reshape(_N, 1), weight)
    return (feat, out)


# speedup vs baseline: 18.4839x; 1.1014x over previous
import math

import jax
import jax.numpy as jnp
from jax import lax
from jax.experimental import pallas as pl
from jax.experimental.pallas import tpu as pltpu
from jax.experimental.pallas import tpu_sc as plsc

_N = 10000
_E = 320000
_D_IN = 128
_D_OUT = 16
_N_LABELS = 32
_S = 64.0
_M = 0.1
_COS_M = math.cos(_M)
_SIN_M = math.sin(_M)
_TH = math.cos(math.pi - _M)
_MM = math.sin(math.pi - _M) * _M

_NC = 2
_NS = 16
_NW = _NC * _NS
_EPW = _E // _NW
_C = 80
_K = _EPW // _C
_NBUF = 5
_G = _K // _NBUF
_RPS = _N // _NS

_ROWS_TC = 2000


def _project(x, wl, wr, bl2, br2):
    grid = (_N // _ROWS_TC,)

    def body(x_ref, wl_ref, wr_ref, bl_ref, br_ref, y_ref, z_ref):
        xb = x_ref[...]
        dn = (((1,), (1,)), ((), ()))
        y_ref[...] = lax.dot_general(xb, wl_ref[...], dn,
                                     preferred_element_type=jnp.float32)
        z_ref[...] = (
            lax.dot_general(xb, wr_ref[...], dn,
                            preferred_element_type=jnp.float32)
            + bl_ref[...] + br_ref[...]
        )

    return pl.pallas_call(
        body,
        grid=grid,
        in_specs=[
            pl.BlockSpec((_ROWS_TC, _D_IN), lambda i: (i, 0)),
            pl.BlockSpec((_D_OUT, _D_IN), lambda i: (0, 0)),
            pl.BlockSpec((_D_OUT, _D_IN), lambda i: (0, 0)),
            pl.BlockSpec((1, _D_OUT), lambda i: (0, 0)),
            pl.BlockSpec((1, _D_OUT), lambda i: (0, 0)),
        ],
        out_specs=[
            pl.BlockSpec((_ROWS_TC, _D_OUT), lambda i: (i, 0)),
            pl.BlockSpec((_ROWS_TC, _D_OUT), lambda i: (i, 0)),
        ],
        out_shape=[
            jax.ShapeDtypeStruct((_N, _D_OUT), jnp.float32),
            jax.ShapeDtypeStruct((_N, _D_OUT), jnp.float32),
        ],
    )(x, wl, wr, bl2, br2)


def _segment_sum_sc(y, ei):
    mesh = plsc.VectorSubcoreMesh(
        core_axis_name="c", subcore_axis_name="s",
        num_cores=_NC, num_subcores=_NS,
    )

    @pl.kernel(
        out_type=[
            jax.ShapeDtypeStruct((_NC, _N, _D_OUT), jnp.float32),
            jax.ShapeDtypeStruct((_NC, _N, _D_OUT), jnp.float32),
        ],
        mesh=mesh,
        scratch_types=[
            pltpu.VMEM((_K, _C), jnp.int32),
            pltpu.VMEM((_K, _C), jnp.int32),
            pltpu.VMEM((_NBUF, _C, _D_OUT), jnp.float32),
            pltpu.VMEM((_C, _D_OUT), jnp.float32),
            pltpu.VMEM((_RPS, _D_OUT), jnp.float32),
            pltpu.VMEM_SHARED((_N, _D_OUT), jnp.float32),
            pltpu.VMEM_SHARED((_N, _D_OUT), jnp.float32),
            pltpu.SemaphoreType.DMA,
            pltpu.SemaphoreType.DMA,
        ],
        compiler_params=pltpu.CompilerParams(use_tc_tiling_on_sc=False),
    )
    def seg(y_hbm, ei_hbm, agg_out, cnt_out,
            src_v, dst_v, rows_v, ones_v, tmp_v, agg_s, cnt_s, gsem, ssem):
        cid = lax.axis_index("c")
        sid = lax.axis_index("s")
        w = cid * _NS + sid

        ones16 = jnp.ones((_D_OUT,), jnp.float32)
        zeros16 = jnp.zeros((_D_OUT,), jnp.float32)

        def fill_ones(i, carry):
            ones_v[i, :] = ones16
            return carry
        lax.fori_loop(0, _C, fill_ones, 0)

        def fill_zero(i, carry):
            tmp_v[i, :] = zeros16
            return carry
        lax.fori_loop(0, _RPS, fill_zero, 0)

        row0 = sid * _RPS
        pltpu.sync_copy(tmp_v, agg_s.at[pl.ds(row0, _RPS)])
        pltpu.sync_copy(tmp_v, cnt_s.at[pl.ds(row0, _RPS)])

        pltpu.sync_copy(ei_hbm.at[0].at[w], src_v)
        pltpu.sync_copy(ei_hbm.at[1].at[w], dst_v)

        plsc.subcore_barrier()

        def group(g, carry):
            base = g * _NBUF
            descs = []
            for b in range(_NBUF):
                descs.append(
                    pltpu.async_copy(y_hbm.at[src_v.at[base + b]],
                                     rows_v.at[b], gsem))
            for d in descs:
                d.wait()
            sdescs = []
            for b in range(_NBUF):
                sdescs.append(
                    pltpu.async_copy(rows_v.at[b],
                                     agg_s.at[dst_v.at[base + b]],
                                     ssem, add=True))
                sdescs.append(
                    pltpu.async_copy(ones_v,
                                     cnt_s.at[dst_v.at[base + b]],
                                     ssem, add=True))
            for d in sdescs:
                d.wait()
            return carry
        lax.fori_loop(0, _G, group, 0)

        plsc.subcore_barrier()

        pltpu.sync_copy(agg_s.at[pl.ds(row0, _RPS)], tmp_v)
        pltpu.sync_copy(tmp_v, agg_out.at[cid].at[pl.ds(row0, _RPS)])
        pltpu.sync_copy(cnt_s.at[pl.ds(row0, _RPS)], tmp_v)
        pltpu.sync_copy(tmp_v, cnt_out.at[cid].at[pl.ds(row0, _RPS)])

    return seg(y, ei)


def _head(agg, cnt, z, label2, weight):
    grid = (_N // _ROWS_TC,)

    def body(a_ref, c_ref, z_ref, l_ref, w_ref, feat_ref, out_ref):
        aggs = a_ref[0] + a_ref[1]
        cnts = c_ref[0, :, 0:1] + c_ref[1, :, 0:1]
        mean = aggs / jnp.maximum(cnts, 1.0)
        h = mean + z_ref[...]
        feat_ref[...] = h
        hr = jnp.maximum(h, 0.0)
        nrm = jnp.sqrt(jnp.sum(hr * hr, axis=1, keepdims=True))
        xn = hr / jnp.maximum(nrm, 1e-12)
        wv = w_ref[...]
        wn = wv / jnp.maximum(
            jnp.sqrt(jnp.sum(wv * wv, axis=1, keepdims=True)), 1e-12)
        cos = lax.dot_general(xn, wn, (((1,), (1,)), ((), ())),
                              preferred_element_type=jnp.float32)
        sine = jnp.sqrt(jnp.clip(1.0 - cos * cos, 0.0, 1.0))
        phi = cos * _COS_M - sine * _SIN_M
        phi = jnp.where(cos > _TH, phi, cos - _MM)
        onehot = lax.broadcasted_iota(jnp.int32, cos.shape, 1) == l_ref[...]
        out_ref[...] = jnp.where(onehot, phi, cos) * _S

    return pl.pallas_call(
        body,
        grid=grid,
        in_specs=[
            pl.BlockSpec((_NC, _ROWS_TC, _D_OUT), lambda i: (0, i, 0)),
            pl.BlockSpec((_NC, _ROWS_TC, _D_OUT), lambda i: (0, i, 0)),
            pl.BlockSpec((_ROWS_TC, _D_OUT), lambda i: (i, 0)),
            pl.BlockSpec((_ROWS_TC, 1), lambda i: (i, 0)),
            pl.BlockSpec((_N_LABELS, _D_OUT), lambda i: (0, 0)),
        ],
        out_specs=[
            pl.BlockSpec((_ROWS_TC, _D_OUT), lambda i: (i, 0)),
            pl.BlockSpec((_ROWS_TC, _N_LABELS), lambda i: (i, 0)),
        ],
        out_shape=[
            jax.ShapeDtypeStruct((_N, _D_OUT), jnp.float32),
            jax.ShapeDtypeStruct((_N, _N_LABELS), jnp.float32),
        ],
    )(agg, cnt, z, label2, weight)


def kernel(x, edge_index, label, Wl, bl, Wr, br, weight):
    y, z = _project(x, Wl, Wr, bl.reshape(1, _D_OUT), br.reshape(1, _D_OUT))
    ei = edge_index.reshape(2, _NW, _K, _C)
    agg, cnt = _segment_sum_sc(y, ei)
    feat, out = _head(agg, cnt, z, label.reshape(_N, 1), weight)
    return (feat, out)


# speedup vs baseline: 20.5018x; 1.1092x over previous
import math

import jax
import jax.numpy as jnp
from jax import lax
from jax.experimental import pallas as pl
from jax.experimental.pallas import tpu as pltpu
from jax.experimental.pallas import tpu_sc as plsc

_N = 10000
_E = 320000
_D_IN = 128
_D_OUT = 16
_N_LABELS = 32
_S = 64.0
_M = 0.1
_COS_M = math.cos(_M)
_SIN_M = math.sin(_M)
_TH = math.cos(math.pi - _M)
_MM = math.sin(math.pi - _M) * _M

_NC = 2
_NS = 16
_NW = _NC * _NS
_EPW = _E // _NW
_C = 80
_K = _EPW // _C
_NBUF = 25
_G = _K // _NBUF
_RPS = _N // _NS

_ROWS_TC = 2000


def _project(x, wl, wr, bl2, br2):
    grid = (_N // _ROWS_TC,)

    def body(x_ref, wl_ref, wr_ref, bl_ref, br_ref, y_ref, z_ref):
        xb = x_ref[...]
        dn = (((1,), (1,)), ((), ()))
        y_ref[...] = lax.dot_general(xb, wl_ref[...], dn,
                                     preferred_element_type=jnp.float32)
        z_ref[...] = (
            lax.dot_general(xb, wr_ref[...], dn,
                            preferred_element_type=jnp.float32)
            + bl_ref[...] + br_ref[...]
        )

    return pl.pallas_call(
        body,
        grid=grid,
        in_specs=[
            pl.BlockSpec((_ROWS_TC, _D_IN), lambda i: (i, 0)),
            pl.BlockSpec((_D_OUT, _D_IN), lambda i: (0, 0)),
            pl.BlockSpec((_D_OUT, _D_IN), lambda i: (0, 0)),
            pl.BlockSpec((1, _D_OUT), lambda i: (0, 0)),
            pl.BlockSpec((1, _D_OUT), lambda i: (0, 0)),
        ],
        out_specs=[
            pl.BlockSpec((_ROWS_TC, _D_OUT), lambda i: (i, 0)),
            pl.BlockSpec((_ROWS_TC, _D_OUT), lambda i: (i, 0)),
        ],
        out_shape=[
            jax.ShapeDtypeStruct((_N, _D_OUT), jnp.float32),
            jax.ShapeDtypeStruct((_N, _D_OUT), jnp.float32),
        ],
    )(x, wl, wr, bl2, br2)


def _segment_sum_sc(y, ei):
    mesh = plsc.VectorSubcoreMesh(
        core_axis_name="c", subcore_axis_name="s",
        num_cores=_NC, num_subcores=_NS,
    )

    @pl.kernel(
        out_type=[
            jax.ShapeDtypeStruct((_NC, _N, _D_OUT), jnp.float32),
            jax.ShapeDtypeStruct((_NC, _N, _D_OUT), jnp.float32),
        ],
        mesh=mesh,
        scratch_types=[
            pltpu.VMEM((_K, _C), jnp.int32),
            pltpu.VMEM((_K, _C), jnp.int32),
            pltpu.VMEM((_NBUF, _C, _D_OUT), jnp.float32),
            pltpu.VMEM((_C, _D_OUT), jnp.float32),
            pltpu.VMEM((_RPS, _D_OUT), jnp.float32),
            pltpu.VMEM_SHARED((_N, _D_OUT), jnp.float32),
            pltpu.VMEM_SHARED((_N, _D_OUT), jnp.float32),
            pltpu.SemaphoreType.DMA,
            pltpu.SemaphoreType.DMA,
        ],
        compiler_params=pltpu.CompilerParams(use_tc_tiling_on_sc=False),
    )
    def seg(y_hbm, ei_hbm, agg_out, cnt_out,
            src_v, dst_v, rows_v, ones_v, tmp_v, agg_s, cnt_s, gsem, ssem):
        cid = lax.axis_index("c")
        sid = lax.axis_index("s")
        w = cid * _NS + sid

        ones16 = jnp.ones((_D_OUT,), jnp.float32)
        zeros16 = jnp.zeros((_D_OUT,), jnp.float32)

        def fill_ones(i, carry):
            ones_v[i, :] = ones16
            return carry
        lax.fori_loop(0, _C, fill_ones, 0)

        def fill_zero(i, carry):
            tmp_v[i, :] = zeros16
            return carry
        lax.fori_loop(0, _RPS, fill_zero, 0)

        row0 = sid * _RPS
        pltpu.sync_copy(tmp_v, agg_s.at[pl.ds(row0, _RPS)])
        pltpu.sync_copy(tmp_v, cnt_s.at[pl.ds(row0, _RPS)])

        pltpu.sync_copy(ei_hbm.at[0].at[w], src_v)
        pltpu.sync_copy(ei_hbm.at[1].at[w], dst_v)

        plsc.subcore_barrier()

        def group(g, carry):
            base = g * _NBUF
            descs = []
            for b in range(_NBUF):
                descs.append(
                    pltpu.async_copy(y_hbm.at[src_v.at[base + b]],
                                     rows_v.at[b], gsem))
            for d in descs:
                d.wait()
            sdescs = []
            for b in range(_NBUF):
                sdescs.append(
                    pltpu.async_copy(rows_v.at[b],
                                     agg_s.at[dst_v.at[base + b]],
                                     ssem, add=True))
                sdescs.append(
                    pltpu.async_copy(ones_v,
                                     cnt_s.at[dst_v.at[base + b]],
                                     ssem, add=True))
            for d in sdescs:
                d.wait()
            return carry
        lax.fori_loop(0, _G, group, 0)

        plsc.subcore_barrier()

        pltpu.sync_copy(agg_s.at[pl.ds(row0, _RPS)], tmp_v)
        pltpu.sync_copy(tmp_v, agg_out.at[cid].at[pl.ds(row0, _RPS)])
        pltpu.sync_copy(cnt_s.at[pl.ds(row0, _RPS)], tmp_v)
        pltpu.sync_copy(tmp_v, cnt_out.at[cid].at[pl.ds(row0, _RPS)])

    return seg(y, ei)


def _head(agg, cnt, z, label2, weight):
    grid = (_N // _ROWS_TC,)

    def body(a_ref, c_ref, z_ref, l_ref, w_ref, feat_ref, out_ref):
        aggs = a_ref[0] + a_ref[1]
        cnts = c_ref[0, :, 0:1] + c_ref[1, :, 0:1]
        mean = aggs / jnp.maximum(cnts, 1.0)
        h = mean + z_ref[...]
        feat_ref[...] = h
        hr = jnp.maximum(h, 0.0)
        nrm = jnp.sqrt(jnp.sum(hr * hr, axis=1, keepdims=True))
        xn = hr / jnp.maximum(nrm, 1e-12)
        wv = w_ref[...]
        wn = wv / jnp.maximum(
            jnp.sqrt(jnp.sum(wv * wv, axis=1, keepdims=True)), 1e-12)
        cos = lax.dot_general(xn, wn, (((1,), (1,)), ((), ())),
                              preferred_element_type=jnp.float32)
        sine = jnp.sqrt(jnp.clip(1.0 - cos * cos, 0.0, 1.0))
        phi = cos * _COS_M - sine * _SIN_M
        phi = jnp.where(cos > _TH, phi, cos - _MM)
        onehot = lax.broadcasted_iota(jnp.int32, cos.shape, 1) == l_ref[...]
        out_ref[...] = jnp.where(onehot, phi, cos) * _S

    return pl.pallas_call(
        body,
        grid=grid,
        in_specs=[
            pl.BlockSpec((_NC, _ROWS_TC, _D_OUT), lambda i: (0, i, 0)),
            pl.BlockSpec((_NC, _ROWS_TC, _D_OUT), lambda i: (0, i, 0)),
            pl.BlockSpec((_ROWS_TC, _D_OUT), lambda i: (i, 0)),
            pl.BlockSpec((_ROWS_TC, 1), lambda i: (i, 0)),
            pl.BlockSpec((_N_LABELS, _D_OUT), lambda i: (0, 0)),
        ],
        out_specs=[
            pl.BlockSpec((_ROWS_TC, _D_OUT), lambda i: (i, 0)),
            pl.BlockSpec((_ROWS_TC, _N_LABELS), lambda i: (i, 0)),
        ],
        out_shape=[
            jax.ShapeDtypeStruct((_N, _D_OUT), jnp.float32),
            jax.ShapeDtypeStruct((_N, _N_LABELS), jnp.float32),
        ],
    )(agg, cnt, z, label2, weight)


def kernel(x, edge_index, label, Wl, bl, Wr, br, weight):
    y, z = _project(x, Wl, Wr, bl.reshape(1, _D_OUT), br.reshape(1, _D_OUT))
    ei = edge_index.reshape(2, _NW, _K, _C)
    agg, cnt = _segment_sum_sc(y, ei)
    feat, out = _head(agg, cnt, z, label.reshape(_N, 1), weight)
    return (feat, out)


# speedup vs baseline: 22.5450x; 1.0997x over previous
import math

import jax
import jax.numpy as jnp
from jax import lax
from jax.experimental import pallas as pl
from jax.experimental.pallas import tpu as pltpu
from jax.experimental.pallas import tpu_sc as plsc

_N = 10000
_E = 320000
_D_IN = 128
_D_OUT = 16
_N_LABELS = 32
_S = 64.0
_M = 0.1
_COS_M = math.cos(_M)
_SIN_M = math.sin(_M)
_TH = math.cos(math.pi - _M)
_MM = math.sin(math.pi - _M) * _M

_NC = 2
_NS = 16
_NW = _NC * _NS
_EPW = _E // _NW
_C = 80
_K = _EPW // _C
_NBUF = 25
_G = _K // _NBUF
_RPS = _N // _NS

_ROWS_TC = 2000


def _project(x, wl, wr, bl2, br2):
    grid = (_N // _ROWS_TC,)

    def body(x_ref, wl_ref, wr_ref, bl_ref, br_ref, y_ref, z_ref):
        xb = x_ref[...]
        dn = (((1,), (1,)), ((), ()))
        y_ref[...] = lax.dot_general(xb, wl_ref[...], dn,
                                     preferred_element_type=jnp.float32)
        z_ref[...] = (
            lax.dot_general(xb, wr_ref[...], dn,
                            preferred_element_type=jnp.float32)
            + bl_ref[...] + br_ref[...]
        )

    return pl.pallas_call(
        body,
        grid=grid,
        in_specs=[
            pl.BlockSpec((_ROWS_TC, _D_IN), lambda i: (i, 0)),
            pl.BlockSpec((_D_OUT, _D_IN), lambda i: (0, 0)),
            pl.BlockSpec((_D_OUT, _D_IN), lambda i: (0, 0)),
            pl.BlockSpec((1, _D_OUT), lambda i: (0, 0)),
            pl.BlockSpec((1, _D_OUT), lambda i: (0, 0)),
        ],
        out_specs=[
            pl.BlockSpec((_ROWS_TC, _D_OUT), lambda i: (i, 0)),
            pl.BlockSpec((_ROWS_TC, _D_OUT), lambda i: (i, 0)),
        ],
        out_shape=[
            jax.ShapeDtypeStruct((_N, _D_OUT), jnp.float32),
            jax.ShapeDtypeStruct((_N, _D_OUT), jnp.float32),
        ],
    )(x, wl, wr, bl2, br2)


def _segment_sum_sc(y, ei):
    mesh = plsc.VectorSubcoreMesh(
        core_axis_name="c", subcore_axis_name="s",
        num_cores=_NC, num_subcores=_NS,
    )

    @pl.kernel(
        out_type=[
            jax.ShapeDtypeStruct((_NC, _N, _D_OUT), jnp.float32),
            jax.ShapeDtypeStruct((_NC, _N, _D_OUT), jnp.float32),
        ],
        mesh=mesh,
        scratch_types=[
            pltpu.VMEM((_K, _C), jnp.int32),
            pltpu.VMEM((_K, _C), jnp.int32),
            pltpu.VMEM((2, _NBUF, _C, _D_OUT), jnp.float32),
            pltpu.VMEM((_C, _D_OUT), jnp.float32),
            pltpu.VMEM((_RPS, _D_OUT), jnp.float32),
            pltpu.VMEM_SHARED((_N, _D_OUT), jnp.float32),
            pltpu.VMEM_SHARED((_N, _D_OUT), jnp.float32),
            pltpu.SemaphoreType.DMA,
            pltpu.SemaphoreType.DMA,
            pltpu.SemaphoreType.DMA,
        ],
        compiler_params=pltpu.CompilerParams(use_tc_tiling_on_sc=False),
    )
    def seg(y_hbm, ei_hbm, agg_out, cnt_out,
            src_v, dst_v, rows_v, ones_v, tmp_v, agg_s, cnt_s,
            gsem0, gsem1, ssem):
        cid = lax.axis_index("c")
        sid = lax.axis_index("s")
        w = cid * _NS + sid

        ones16 = jnp.ones((_D_OUT,), jnp.float32)
        zeros16 = jnp.zeros((_D_OUT,), jnp.float32)

        def fire_gathers(base, bank, sem):
            for i in range(_NBUF):
                pltpu.async_copy(y_hbm.at[src_v.at[base + i]],
                                 rows_v.at[bank].at[i], sem)

        def drain_gathers(bank, sem):
            for i in range(_NBUF):
                pltpu.make_async_copy(y_hbm.at[src_v.at[i]],
                                      rows_v.at[bank].at[i], sem).wait()

        def do_scatters(base, bank):
            descs = []
            for i in range(_NBUF):
                descs.append(
                    pltpu.async_copy(rows_v.at[bank].at[i],
                                     agg_s.at[dst_v.at[base + i]],
                                     ssem, add=True))
                descs.append(
                    pltpu.async_copy(ones_v,
                                     cnt_s.at[dst_v.at[base + i]],
                                     ssem, add=True))
            for d in descs:
                d.wait()

        pltpu.sync_copy(ei_hbm.at[0].at[w], src_v)
        pltpu.sync_copy(ei_hbm.at[1].at[w], dst_v)

        fire_gathers(0, 0, gsem0)

        def fill_ones(i, carry):
            ones_v[i, :] = ones16
            return carry
        lax.fori_loop(0, _C, fill_ones, 0)

        def fill_zero(i, carry):
            tmp_v[i, :] = zeros16
            return carry
        lax.fori_loop(0, _RPS, fill_zero, 0)

        row0 = sid * _RPS
        pltpu.sync_copy(tmp_v, agg_s.at[pl.ds(row0, _RPS)])
        pltpu.sync_copy(tmp_v, cnt_s.at[pl.ds(row0, _RPS)])

        plsc.subcore_barrier()

        def group(g, carry):
            @pl.when(g % 2 == 0)
            def _():
                drain_gathers(0, gsem0)

                @pl.when(g + 1 < _G)
                def _():
                    fire_gathers((g + 1) * _NBUF, 1, gsem1)
                do_scatters(g * _NBUF, 0)

            @pl.when(g % 2 == 1)
            def _():
                drain_gathers(1, gsem1)

                @pl.when(g + 1 < _G)
                def _():
                    fire_gathers((g + 1) * _NBUF, 0, gsem0)
                do_scatters(g * _NBUF, 1)
            return carry
        lax.fori_loop(0, _G, group, 0)

        plsc.subcore_barrier()

        pltpu.sync_copy(agg_s.at[pl.ds(row0, _RPS)], tmp_v)
        pltpu.sync_copy(tmp_v, agg_out.at[cid].at[pl.ds(row0, _RPS)])
        pltpu.sync_copy(cnt_s.at[pl.ds(row0, _RPS)], tmp_v)
        pltpu.sync_copy(tmp_v, cnt_out.at[cid].at[pl.ds(row0, _RPS)])

    return seg(y, ei)


def _head(agg, cnt, z, label2, weight):
    grid = (_N // _ROWS_TC,)

    def body(a_ref, c_ref, z_ref, l_ref, w_ref, feat_ref, out_ref):
        aggs = a_ref[0] + a_ref[1]
        cnts = c_ref[0, :, 0:1] + c_ref[1, :, 0:1]
        mean = aggs / jnp.maximum(cnts, 1.0)
        h = mean + z_ref[...]
        feat_ref[...] = h
        hr = jnp.maximum(h, 0.0)
        nrm = jnp.sqrt(jnp.sum(hr * hr, axis=1, keepdims=True))
        xn = hr / jnp.maximum(nrm, 1e-12)
        wv = w_ref[...]
        wn = wv / jnp.maximum(
            jnp.sqrt(jnp.sum(wv * wv, axis=1, keepdims=True)), 1e-12)
        cos = lax.dot_general(xn, wn, (((1,), (1,)), ((), ())),
                              preferred_element_type=jnp.float32)
        sine = jnp.sqrt(jnp.clip(1.0 - cos * cos, 0.0, 1.0))
        phi = cos * _COS_M - sine * _SIN_M
        phi = jnp.where(cos > _TH, phi, cos - _MM)
        onehot = lax.broadcasted_iota(jnp.int32, cos.shape, 1) == l_ref[...]
        out_ref[...] = jnp.where(onehot, phi, cos) * _S

    return pl.pallas_call(
        body,
        grid=grid,
        in_specs=[
            pl.BlockSpec((_NC, _ROWS_TC, _D_OUT), lambda i: (0, i, 0)),
            pl.BlockSpec((_NC, _ROWS_TC, _D_OUT), lambda i: (0, i, 0)),
            pl.BlockSpec((_ROWS_TC, _D_OUT), lambda i: (i, 0)),
            pl.BlockSpec((_ROWS_TC, 1), lambda i: (i, 0)),
            pl.BlockSpec((_N_LABELS, _D_OUT), lambda i: (0, 0)),
        ],
        out_specs=[
            pl.BlockSpec((_ROWS_TC, _D_OUT), lambda i: (i, 0)),
            pl.BlockSpec((_ROWS_TC, _N_LABELS), lambda i: (i, 0)),
        ],
        out_shape=[
            jax.ShapeDtypeStruct((_N, _D_OUT), jnp.float32),
            jax.ShapeDtypeStruct((_N, _N_LABELS), jnp.float32),
        ],
    )(agg, cnt, z, label2, weight)


def kernel(x, edge_index, label, Wl, bl, Wr, br, weight):
    y, z = _project(x, Wl, Wr, bl.reshape(1, _D_OUT), br.reshape(1, _D_OUT))
    ei = edge_index.reshape(2, _NW, _K, _C)
    agg, cnt = _segment_sum_sc(y, ei)
    feat, out = _head(agg, cnt, z, label.reshape(_N, 1), weight)
    return (feat, out)
